# baseline (speedup 1.0000x reference)
import dataclasses
import functools

import jax
import jax.numpy as jnp
from jax import lax
from jax.experimental import pallas as pl
from jax.experimental.pallas import tpu as pltpu
from jax.experimental.pallas import tpu_sc as plsc

_N = 10000
_E = 320000
_D = 128
_HID = 128
_HEADS = 8
_OUT = 128
_G = 16

_NC = 2
_NS = 16
_NW = _NC * _NS
_B = 80
_NP = 10112
_EPW = _E // _NW
_NCH = _EPW // _B
_DCH = _NP // 128


def _mesh():
    return plsc.VectorSubcoreMesh(core_axis_name="c", subcore_axis_name="s")


def _sc_params():
    cp = pltpu.CompilerParams()
    if "needs_layout_passes" in pltpu.CompilerParams.__dataclass_fields__:
        cp = dataclasses.replace(cp, needs_layout_passes=False)
    return cp


def _wid():
    return lax.axis_index("s") * _NC + lax.axis_index("c")


def _al8(v):
    return pl.multiple_of(v, 8)


def _attn_body(heads, C, xl_hbm, xr_hbm, srcp_hbm, dstp_hbm, att_hbm,
               alpha_hbm, part_hbm,
               src_all, dst_all, il0, ir0, il1, ir1, xl0, xr0, xl1, xr1,
               al_all, att_v, ath_v, loc_v, sem0, sem1):
    nsl = C // 16
    wid = _wid()
    ebase = wid * _EPW
    pltpu.sync_copy(att_hbm, att_v)
    pltpu.sync_copy(srcp_hbm.at[pl.ds(_al8(ebase), _NP)], src_all)
    pltpu.sync_copy(dstp_hbm.at[pl.ds(_al8(ebase), _NP)], dst_all)
    lane0 = lax.iota(jnp.int32, 16) == 0
    ci0 = lax.iota(jnp.int32, 16)

    @pl.loop(0, heads)
    def _h(h):
        for cs in range(nsl):
            ath_v[pl.ds(cs * 16, 16)] = att_v[pl.ds(_al8(h * C) + cs * 16, 16)]

        @pl.loop(0, _NP, step=16)
        def _z(i):
            loc_v[pl.ds(i, 16)] = jnp.full((16,), -jnp.inf, jnp.float32)

        def fire(ch, il, ir, xl, xr, sem):
            base = ch * _B

            @pl.loop(0, _B, step=16)
            def _i(i):
                sl = pl.ds(base + i, 16)
                il[pl.ds(i, 16)] = src_all[sl] * heads + h
                ir[pl.ds(i, 16)] = dst_all[sl] * heads + h

            pltpu.async_copy(xl_hbm.at[il], xl, sem)
            pltpu.async_copy(xr_hbm.at[ir], xr, sem)

        def crunch(ch, il, ir, xl, xr, sem):
            pltpu.make_async_copy(xl_hbm.at[il], xl, sem).wait()
            pltpu.make_async_copy(xr_hbm.at[ir], xr, sem).wait()
            base = ch * _B

            @pl.loop(0, _B)
            def _e(e):
                er = jnp.full((16,), e, jnp.int32)
                acc = jnp.zeros((16,), jnp.float32)
                for cs in range(nsl):
                    ci = ci0 + cs * 16
                    t = (plsc.load_gather(xl, [er, ci])
                         + plsc.load_gather(xr, [er, ci]))
                    t = jnp.maximum(t, 0.2 * t)
                    acc = acc + t * ath_v[pl.ds(cs * 16, 16)]
                s = jnp.sum(acc)
                plsc.store_scatter(al_all,
                                   [jnp.full((16,), base + e, jnp.int32)],
                                   jnp.full((16,), s, jnp.float32),
                                   mask=lane0)

            @pl.loop(0, _B, step=16)
            def _m(i):
                sl = pl.ds(base + i, 16)
                idx = dst_all[sl]
                val = al_all[sl]
                cur = plsc.load_gather(loc_v, [idx])
                plsc.store_scatter(loc_v, [idx], jnp.maximum(cur, val))
                chk = plsc.load_gather(loc_v, [idx])

                @pl.when(jnp.any(chk < val))
                def _fix():
                    for _ in range(15):
                        c2 = plsc.load_gather(loc_v, [idx])
                        plsc.store_scatter(loc_v, [idx],
                                           jnp.maximum(c2, val),
                                           mask=c2 < val)

        fire(0, il0, ir0, xl0, xr0, sem0)

        @pl.loop(0, _NCH - 1, step=2)
        def _p(ch):
            fire(ch + 1, il1, ir1, xl1, xr1, sem1)
            crunch(ch, il0, ir0, xl0, xr0, sem0)
            fire(ch + 2, il0, ir0, xl0, xr0, sem0)
            crunch(ch + 1, il1, ir1, xl1, xr1, sem1)

        crunch(_NCH - 1, il0, ir0, xl0, xr0, sem0)

        off = _al8((h * _NW + wid) * _NP)
        pltpu.sync_copy(al_all, alpha_hbm.at[pl.ds(off, _NP)])
        pltpu.sync_copy(loc_v, part_hbm.at[pl.ds(off, _NP)])


def _run_attn(heads, xl2d, xr2d, srcp, dstp, att_flat):
    C = xl2d.shape[1]
    kern = pl.kernel(
        functools.partial(_attn_body, heads, C),
        out_type=[jax.ShapeDtypeStruct((heads * _NW * _NP,), jnp.float32),
                  jax.ShapeDtypeStruct((heads * _NW * _NP,), jnp.float32)],
        mesh=_mesh(),
        compiler_params=_sc_params(),
        scratch_types=[
            pltpu.VMEM((_NP,), jnp.int32),
            pltpu.VMEM((_NP,), jnp.int32),
            pltpu.VMEM((_B,), jnp.int32),
            pltpu.VMEM((_B,), jnp.int32),
            pltpu.VMEM((_B,), jnp.int32),
            pltpu.VMEM((_B,), jnp.int32),
            pltpu.VMEM((_B, C), jnp.float32),
            pltpu.VMEM((_B, C), jnp.float32),
            pltpu.VMEM((_B, C), jnp.float32),
            pltpu.VMEM((_B, C), jnp.float32),
            pltpu.VMEM((_NP,), jnp.float32),
            pltpu.VMEM((heads * C,), jnp.float32),
            pltpu.VMEM((C,), jnp.float32),
            pltpu.VMEM((_NP,), jnp.float32),
            pltpu.SemaphoreType.DMA,
            pltpu.SemaphoreType.DMA,
        ],
    )
    return kern(xl2d, xr2d, srcp, dstp, att_flat)


def _attn_pair_body(heads, C, xl_hbm, xr_hbm, srcp_hbm, dstp_hbm, att_hbm,
                    alpha_hbm, part_hbm,
                    src_all, dst_all, il0, ir0, il1, ir1, xl0, xr0, xl1, xr1,
                    al_a, al_b, att_v, ath_v, loc_a, loc_b, sem0, sem1):
    hp_n = heads // 2
    ns4 = C // 32
    wid = _wid()
    ebase = wid * _EPW
    pltpu.sync_copy(att_hbm, att_v)
    pltpu.sync_copy(srcp_hbm.at[pl.ds(_al8(ebase), _NP)], src_all)
    pltpu.sync_copy(dstp_hbm.at[pl.ds(_al8(ebase), _NP)], dst_all)
    lane0 = lax.iota(jnp.int32, 16) == 0
    ci0 = lax.iota(jnp.int32, 16)

    @pl.loop(0, hp_n)
    def _h(hp):
        for cs in range(2 * C // 16):
            ath_v[pl.ds(cs * 16, 16)] = att_v[
                pl.ds(_al8(hp * 2 * C) + cs * 16, 16)]

        @pl.loop(0, _NP, step=16)
        def _z(i):
            ninf = jnp.full((16,), -jnp.inf, jnp.float32)
            loc_a[pl.ds(i, 16)] = ninf
            loc_b[pl.ds(i, 16)] = ninf

        def fire(ch, il, ir, xl, xr, sem):
            base = ch * _B

            @pl.loop(0, _B, step=16)
            def _i(i):
                sl = pl.ds(base + i, 16)
                il[pl.ds(i, 16)] = src_all[sl] * hp_n + hp
                ir[pl.ds(i, 16)] = dst_all[sl] * hp_n + hp

            pltpu.async_copy(xl_hbm.at[il], xl, sem)
            pltpu.async_copy(xr_hbm.at[ir], xr, sem)

        def crunch(ch, il, ir, xl, xr, sem):
            pltpu.make_async_copy(xl_hbm.at[il], xl, sem).wait()
            pltpu.make_async_copy(xr_hbm.at[ir], xr, sem).wait()
            base = ch * _B

            @pl.loop(0, _B)
            def _e(e):
                er = jnp.full((16,), e, jnp.int32)
                acc_a = jnp.zeros((16,), jnp.float32)
                acc_b = jnp.zeros((16,), jnp.float32)
                for cs in range(2 * ns4):
                    ci = ci0 + cs * 16
                    ale, alo = plsc.unpack(
                        plsc.bitcast(plsc.load_gather(xl, [er, ci]),
                                     jnp.bfloat16),
                        format=plsc.PackFormat.INTERLEAVED)
                    are, aro = plsc.unpack(
                        plsc.bitcast(plsc.load_gather(xr, [er, ci]),
                                     jnp.bfloat16),
                        format=plsc.PackFormat.INTERLEAVED)
                    te = ale + are
                    te = jnp.maximum(te, 0.2 * te)
                    to = alo + aro
                    to = jnp.maximum(to, 0.2 * to)
                    if cs < ns4:
                        ae = ath_v[pl.ds(cs * 16, 16)]
                        ao = ath_v[pl.ds(C // 2 + cs * 16, 16)]
                        acc_a = acc_a + te * ae + to * ao
                    else:
                        ae = ath_v[pl.ds(C // 2 + cs * 16, 16)]
                        ao = ath_v[pl.ds(C + cs * 16, 16)]
                        acc_b = acc_b + te * ae + to * ao
                ei = jnp.full((16,), base + e, jnp.int32)
                plsc.store_scatter(al_a, [ei],
                                   jnp.full((16,), jnp.sum(acc_a),
                                            jnp.float32), mask=lane0)
                plsc.store_scatter(al_b, [ei],
                                   jnp.full((16,), jnp.sum(acc_b),
                                            jnp.float32), mask=lane0)

            def smax(loc_v, al_all):
                @pl.loop(0, _B, step=16)
                def _m(i):
                    sl = pl.ds(base + i, 16)
                    idx = dst_all[sl]
                    val = al_all[sl]
                    cur = plsc.load_gather(loc_v, [idx])
                    plsc.store_scatter(loc_v, [idx], jnp.maximum(cur, val))
                    chk = plsc.load_gather(loc_v, [idx])

                    @pl.when(jnp.any(chk < val))
                    def _fix():
                        for _ in range(15):
                            c2 = plsc.load_gather(loc_v, [idx])
                            plsc.store_scatter(loc_v, [idx],
                                               jnp.maximum(c2, val),
                                               mask=c2 < val)

            smax(loc_a, al_a)
            smax(loc_b, al_b)

        fire(0, il0, ir0, xl0, xr0, sem0)

        @pl.loop(0, _NCH - 1, step=2)
        def _p(ch):
            fire(ch + 1, il1, ir1, xl1, xr1, sem1)
            crunch(ch, il0, ir0, xl0, xr0, sem0)
            fire(ch + 2, il0, ir0, xl0, xr0, sem0)
            crunch(ch + 1, il1, ir1, xl1, xr1, sem1)

        crunch(_NCH - 1, il0, ir0, xl0, xr0, sem0)

        offa = _al8(((2 * hp) * _NW + wid) * _NP)
        offb = _al8(((2 * hp + 1) * _NW + wid) * _NP)
        pltpu.sync_copy(al_a, alpha_hbm.at[pl.ds(offa, _NP)])
        pltpu.sync_copy(al_b, alpha_hbm.at[pl.ds(offb, _NP)])
        pltpu.sync_copy(loc_a, part_hbm.at[pl.ds(offa, _NP)])
        pltpu.sync_copy(loc_b, part_hbm.at[pl.ds(offb, _NP)])


def _run_attn_pair(heads, xlp, xrp, srcp, dstp, att_de_flat):
    C = xlp.shape[1]
    kern = pl.kernel(
        functools.partial(_attn_pair_body, heads, C),
        out_type=[jax.ShapeDtypeStruct((heads * _NW * _NP,), jnp.float32),
                  jax.ShapeDtypeStruct((heads * _NW * _NP,), jnp.float32)],
        mesh=_mesh(),
        compiler_params=_sc_params(),
        scratch_types=[
            pltpu.VMEM((_NP,), jnp.int32),
            pltpu.VMEM((_NP,), jnp.int32),
            pltpu.VMEM((_B,), jnp.int32),
            pltpu.VMEM((_B,), jnp.int32),
            pltpu.VMEM((_B,), jnp.int32),
            pltpu.VMEM((_B,), jnp.int32),
            pltpu.VMEM((_B, C), jnp.int32),
            pltpu.VMEM((_B, C), jnp.int32),
            pltpu.VMEM((_B, C), jnp.int32),
            pltpu.VMEM((_B, C), jnp.int32),
            pltpu.VMEM((_NP,), jnp.float32),
            pltpu.VMEM((_NP,), jnp.float32),
            pltpu.VMEM((heads * C,), jnp.float32),
            pltpu.VMEM((2 * C,), jnp.float32),
            pltpu.VMEM((_NP,), jnp.float32),
            pltpu.VMEM((_NP,), jnp.float32),
            pltpu.SemaphoreType.DMA,
            pltpu.SemaphoreType.DMA,
        ],
    )
    return kern(xlp, xrp, srcp, dstp, att_de_flat)


def _exden_body(heads, alpha_hbm, dstp_hbm, amax_hbm, ex_hbm, part_hbm,
                dst_all, al_all, ex_all, amax_v, den_loc, tag_loc):
    wid = _wid()
    ebase = wid * _EPW
    pltpu.sync_copy(dstp_hbm.at[pl.ds(_al8(ebase), _NP)], dst_all)
    lanes = lax.iota(jnp.int32, 16)

    @pl.loop(0, heads)
    def _h(h):
        off = _al8((h * _NW + wid) * _NP)
        pltpu.sync_copy(amax_hbm.at[pl.ds(_al8(h * _NP), _NP)], amax_v)
        pltpu.sync_copy(alpha_hbm.at[pl.ds(off, _NP)], al_all)

        @pl.loop(0, _NP, step=16)
        def _z(i):
            den_loc[pl.ds(i, 16)] = jnp.zeros((16,), jnp.float32)

        @pl.loop(0, _EPW, step=16)
        def _i(i):
            sl = pl.ds(i, 16)
            d16 = dst_all[sl]
            am16 = plsc.load_gather(amax_v, [d16])
            val = jnp.exp(al_all[sl] - am16)
            ex_all[sl] = val
            plsc.store_scatter(tag_loc, [d16], lanes)
            wtag = plsc.load_gather(tag_loc, [d16])
            win = wtag == lanes
            cur = plsc.load_gather(den_loc, [d16])
            plsc.store_scatter(den_loc, [d16], cur + val, mask=win)

            @pl.when(jnp.any(~win))
            def _fix():
                pending = ~win
                for _ in range(15):
                    plsc.store_scatter(tag_loc, [d16], lanes, mask=pending)
                    rt = plsc.load_gather(tag_loc, [d16])
                    w2 = pending & (rt == lanes)
                    c2 = plsc.load_gather(den_loc, [d16])
                    plsc.store_scatter(den_loc, [d16], c2 + val, mask=w2)
                    pending = pending & ~w2

        for t in range(_EPW, _NP, 16):
            ex_all[pl.ds(t, 16)] = jnp.zeros((16,), jnp.float32)

        pltpu.sync_copy(ex_all, ex_hbm.at[pl.ds(off, _NP)])
        pltpu.sync_copy(den_loc, part_hbm.at[pl.ds(off, _NP)])


def _run_exden(heads, alpha, dstp, amax):
    kern = pl.kernel(
        functools.partial(_exden_body, heads),
        out_type=[jax.ShapeDtypeStruct((heads * _NW * _NP,), jnp.float32),
                  jax.ShapeDtypeStruct((heads * _NW * _NP,), jnp.float32)],
        mesh=_mesh(),
        compiler_params=_sc_params(),
        scratch_types=[
            pltpu.VMEM((_NP,), jnp.int32),
            pltpu.VMEM((_NP,), jnp.float32),
            pltpu.VMEM((_NP,), jnp.float32),
            pltpu.VMEM((_NP,), jnp.float32),
            pltpu.VMEM((_NP,), jnp.float32),
            pltpu.VMEM((_NP,), jnp.int32),
        ],
    )
    return kern(alpha, dstp, amax)


def _agg_body(heads, C, xl_hbm, ex_hbm, srcp_hbm, dstp_hbm, zer_hbm,
              out_hbm, src_all, dst_all, ex0, ex1, il0, il1, ds0, ds1,
              xl0, xl1, out_sh, sem0, sem1):
    nsl = C // 16
    wid = _wid()
    ebase = wid * _EPW
    sid = lax.axis_index("s")
    cid = lax.axis_index("c")
    pltpu.sync_copy(srcp_hbm.at[pl.ds(_al8(ebase), _NP)], src_all)
    pltpu.sync_copy(dstp_hbm.at[pl.ds(_al8(ebase), _NP)], dst_all)
    ci0 = lax.iota(jnp.int32, 16)

    @pl.loop(0, heads)
    def _h(h):
        @pl.when(sid == 0)
        def _z():
            pltpu.sync_copy(zer_hbm, out_sh)

        off = _al8((h * _NW + wid) * _NP)
        plsc.subcore_barrier()

        def fire(ch, il, dsb, exb, xl, sem):
            base = ch * _B

            @pl.loop(0, _B, step=16)
            def _i(i):
                sl = pl.ds(base + i, 16)
                il[pl.ds(i, 16)] = src_all[sl] * heads + h
                dsb[pl.ds(i, 16)] = dst_all[sl]

            pltpu.async_copy(ex_hbm.at[pl.ds(off + ch * _B, _B)], exb, sem)
            pltpu.async_copy(xl_hbm.at[il], xl, sem)

        def crunch(ch, il, dsb, exb, xl, sem):
            pltpu.make_async_copy(ex_hbm.at[pl.ds(off, _B)], exb, sem).wait()
            pltpu.make_async_copy(xl_hbm.at[il], xl, sem).wait()
            base = ch * _B

            @pl.loop(0, _B, step=16)
            def _e(i):
                av16 = exb[pl.ds(i, 16)]
                for j in range(16):
                    ae = av16[j]
                    er = jnp.full((16,), i + j, jnp.int32)
                    for cs in range(nsl):
                        ci = ci0 + cs * 16
                        v = plsc.load_gather(xl, [er, ci])
                        plsc.store_scatter(xl, [er, ci], v * ae)

            pltpu.sync_copy(xl, out_sh.at[dsb], add=True)

        fire(0, il0, ds0, ex0, xl0, sem0)

        @pl.loop(0, _NCH - 1, step=2)
        def _p(ch):
            fire(ch + 1, il1, ds1, ex1, xl1, sem1)
            crunch(ch, il0, ds0, ex0, xl0, sem0)
            fire(ch + 2, il0, ds0, ex0, xl0, sem0)
            crunch(ch + 1, il1, ds1, ex1, xl1, sem1)

        crunch(_NCH - 1, il0, ds0, ex0, xl0, sem0)

        plsc.subcore_barrier()

        @pl.when(sid == 0)
        def _w():
            pltpu.sync_copy(out_sh, out_hbm.at[cid, h])

        plsc.subcore_barrier()


def _run_agg(heads, xl2d, ex, srcp, dstp, zeros_nc):
    C = xl2d.shape[1]
    kern = pl.kernel(
        functools.partial(_agg_body, heads, C),
        out_type=jax.ShapeDtypeStruct((_NC, heads, _N, C), jnp.float32),
        mesh=_mesh(),
        compiler_params=_sc_params(),
        scratch_types=[
            pltpu.VMEM((_NP,), jnp.int32),
            pltpu.VMEM((_NP,), jnp.int32),
            pltpu.VMEM((_B,), jnp.float32),
            pltpu.VMEM((_B,), jnp.float32),
            pltpu.VMEM((_B,), jnp.int32),
            pltpu.VMEM((_B,), jnp.int32),
            pltpu.VMEM((_B,), jnp.int32),
            pltpu.VMEM((_B,), jnp.int32),
            pltpu.VMEM((_B, C), jnp.float32),
            pltpu.VMEM((_B, C), jnp.float32),
            pltpu.VMEM_SHARED((_N, C), jnp.float32),
            pltpu.SemaphoreType.DMA,
            pltpu.SemaphoreType.DMA,
        ],
    )
    return kern(xl2d, ex, srcp, dstp, zeros_nc)


def _tc_proj(x, wl, wr, bn):
    n, d = x.shape
    k = wl.shape[1]

    def body(x_ref, wl_ref, wr_ref, ol_ref, or_ref):
        xv = x_ref[...]
        ol_ref[...] = jnp.dot(xv, wl_ref[...],
                              preferred_element_type=jnp.float32)
        or_ref[...] = jnp.dot(xv, wr_ref[...],
                              preferred_element_type=jnp.float32)

    return pl.pallas_call(
        body,
        grid=(n // bn,),
        in_specs=[pl.BlockSpec((bn, d), lambda i: (i, 0)),
                  pl.BlockSpec((d, k), lambda i: (0, 0)),
                  pl.BlockSpec((d, k), lambda i: (0, 0))],
        out_specs=[pl.BlockSpec((bn, k), lambda i: (i, 0)),
                   pl.BlockSpec((bn, k), lambda i: (i, 0))],
        out_shape=[jax.ShapeDtypeStruct((n, k), jnp.float32)] * 2,
    )(x, wl, wr)


def _tc_mid(p0, p1, den3, b3, wl3, wr3, bn):
    heads, n, c = p0.shape
    k2 = wl3.shape[2]

    def body(p0_ref, p1_ref, d_ref, b_ref, wl_ref, wr_ref, ol_ref, or_ref):
        accl = jnp.zeros((bn, k2), jnp.float32)
        accr = jnp.zeros((bn, k2), jnp.float32)
        den = d_ref[...]
        for h in range(heads):
            dh = den[:, h][:, None] + 1e-16
            hv = (p0_ref[h] + p1_ref[h]) / dh + b_ref[h]
            hv = jnp.where(hv > 0, hv, jnp.exp(jnp.minimum(hv, 0.0)) - 1.0)
            accl += jnp.dot(hv, wl_ref[h], preferred_element_type=jnp.float32)
            accr += jnp.dot(hv, wr_ref[h], preferred_element_type=jnp.float32)
        ol_ref[...] = accl
        or_ref[...] = accr

    return pl.pallas_call(
        body,
        grid=(n // bn,),
        in_specs=[pl.BlockSpec((heads, bn, c), lambda i: (0, i, 0)),
                  pl.BlockSpec((heads, bn, c), lambda i: (0, i, 0)),
                  pl.BlockSpec((bn, heads), lambda i: (i, 0)),
                  pl.BlockSpec((heads, 1, c), lambda i: (0, 0, 0)),
                  pl.BlockSpec((heads, c, k2), lambda i: (0, 0, 0)),
                  pl.BlockSpec((heads, c, k2), lambda i: (0, 0, 0))],
        out_specs=[pl.BlockSpec((bn, k2), lambda i: (i, 0)),
                   pl.BlockSpec((bn, k2), lambda i: (i, 0))],
        out_shape=[jax.ShapeDtypeStruct((n, k2), jnp.float32)] * 2,
    )(p0, p1, den3, b3, wl3, wr3)


def _tc_merge(part_flat, heads, op):
    part = part_flat.reshape(heads, _NW, _NP)

    def body(p_ref, o_ref):
        if op == "max":
            o_ref[...] = jnp.max(p_ref[...], axis=1, keepdims=True)
        else:
            o_ref[...] = jnp.sum(p_ref[...], axis=1, keepdims=True)

    out = pl.pallas_call(
        body,
        grid=(heads,),
        in_specs=[pl.BlockSpec((1, _NW, _NP), lambda i: (i, 0, 0))],
        out_specs=pl.BlockSpec((1, 1, _NP), lambda i: (i, 0, 0)),
        out_shape=jax.ShapeDtypeStruct((heads, 1, _NP), jnp.float32),
    )(part)
    return out.reshape(heads * _NP)


def _tc_final(p0, p1, den3, b2, batch3, wfc, bfc, bn):
    n = p0.shape[0]
    nblk = n // bn

    def body(p0_ref, p1_ref, d_ref, b_ref, bt_ref, wfc_ref, bfc_ref,
             logits_ref, prob_ref, acc_ref, cnt_ref):
        i = pl.program_id(0)

        @pl.when(i == 0)
        def _init():
            acc_ref[...] = jnp.zeros_like(acc_ref)
            cnt_ref[...] = jnp.zeros_like(cnt_ref)

        dh = d_ref[...] + 1e-16
        hv = (p0_ref[...] + p1_ref[...]) / dh + b_ref[...]
        hv = jnp.where(hv > 0, hv, jnp.exp(jnp.minimum(hv, 0.0)) - 1.0)
        bt = bt_ref[0, 0, :]
        oh = (bt[:, None] == lax.broadcasted_iota(jnp.int32, (1, _G), 1)
              ).astype(jnp.float32)
        acc_ref[...] += lax.dot_general(
            oh, hv, (((0,), (0,)), ((), ())),
            preferred_element_type=jnp.float32)
        cnt_ref[...] += jnp.sum(oh, axis=0)[:, None]

        @pl.when(i == nblk - 1)
        def _fin():
            pooled = acc_ref[...] / jnp.maximum(cnt_ref[...], 1.0)
            logits = jnp.dot(pooled, wfc_ref[...],
                             preferred_element_type=jnp.float32) + bfc_ref[...]
            m = jnp.max(logits, axis=1, keepdims=True)
            ez = jnp.exp(logits - m)
            prob_ref[...] = ez / jnp.sum(ez, axis=1, keepdims=True)
            logits_ref[...] = logits

    return pl.pallas_call(
        body,
        grid=(nblk,),
        in_specs=[pl.BlockSpec((bn, _OUT), lambda i: (i, 0)),
                  pl.BlockSpec((bn, _OUT), lambda i: (i, 0)),
                  pl.BlockSpec((bn, 1), lambda i: (i, 0)),
                  pl.BlockSpec((1, _OUT), lambda i: (0, 0)),
                  pl.BlockSpec((1, 1, bn), lambda i: (i, 0, 0)),
                  pl.BlockSpec((_OUT, 2), lambda i: (0, 0)),
                  pl.BlockSpec((1, 2), lambda i: (0, 0))],
        out_specs=[pl.BlockSpec((_G, 2), lambda i: (0, 0)),
                   pl.BlockSpec((_G, 2), lambda i: (0, 0))],
        out_shape=[jax.ShapeDtypeStruct((_G, 2), jnp.float32)] * 2,
        scratch_shapes=[pltpu.VMEM((_G, _OUT), jnp.float32),
                        pltpu.VMEM((_G, _OUT), jnp.float32)],
    )(p0, p1, den3, b2, batch3, wfc, bfc)


def _gat_layer(heads, xl2d, xr2d, srcp, dstp, att, zeros_nc,
               xlp=None, xrp=None):
    if xlp is not None:
        alpha, amax_p = _run_attn_pair(heads, xlp, xrp, srcp, dstp,
                                       _deint(att).reshape(-1))
    else:
        alpha, amax_p = _run_attn(heads, xl2d, xr2d, srcp, dstp,
                                  att.reshape(-1))
    amax = _tc_merge(amax_p, heads, "max")
    ex, den_p = _run_exden(heads, alpha, dstp, amax)
    out_p = _run_agg(heads, xl2d, ex, srcp, dstp, zeros_nc)
    den = _tc_merge(den_p, heads, "sum")
    den_t = den.reshape(heads, _NP).T
    return out_p, den_t


def _pack(a):
    b = a.astype(jnp.bfloat16).reshape(-1, _HID, 2)
    return jax.lax.bitcast_convert_type(b, jnp.int32)


def _deint(att):
    return jnp.concatenate([att[:, 0::2], att[:, 1::2]], axis=1).reshape(-1)


def kernel(x, edge_index, batch, Wl1, Wr1, att1, b1, Wl2, Wr2, att2, b2,
           Wfc, bfc):
    src, dst = edge_index[0], edge_index[1]
    srcp = jnp.pad(src, (0, _NP - _EPW))
    dstp = jnp.pad(dst, (0, _NP - _EPW))
    zeros_nc = jnp.zeros((_N, _HID), jnp.float32)

    xl1, xr1 = _tc_proj(x, Wl1, Wr1, 2000)
    xl1g = xl1.reshape(_N * _HEADS, _HID)
    xr1g = xr1.reshape(_N * _HEADS, _HID)
    out1p, den1p = _gat_layer(_HEADS, xl1g, xr1g, srcp, dstp, att1, zeros_nc,
                              xlp=_pack(xl1), xrp=_pack(xr1))

    xl2, xr2 = _tc_mid(out1p[0], out1p[1], den1p,
                       b1.reshape(_HEADS, 1, _HID),
                       Wl2.reshape(_HEADS, _HID, _OUT),
                       Wr2.reshape(_HEADS, _HID, _OUT), 2000)
    out2p, den2p = _gat_layer(1, xl2, xr2, srcp, dstp, att2, zeros_nc)

    logits, y_prob = _tc_final(out2p[0].reshape(_N, _OUT),
                               out2p[1].reshape(_N, _OUT),
                               den2p, b2.reshape(1, -1),
                               batch.reshape(_N // 1000, 1, 1000),
                               Wfc, bfc.reshape(1, 2), 1000)
    return (logits, y_prob)


# speedup vs baseline: 1.8264x; 1.8264x over previous
import dataclasses
import functools

import jax
import jax.numpy as jnp
from jax import lax
from jax.experimental import pallas as pl
from jax.experimental.pallas import tpu as pltpu
from jax.experimental.pallas import tpu_sc as plsc

_N = 10000
_E = 320000
_D = 128
_HID = 128
_HEADS = 8
_OUT = 128
_G = 16

_NC = 2
_NS = 16
_NW = _NC * _NS
_B = 80
_NP = 10112
_EPW = _E // _NW
_NCH = _EPW // _B
_DCH = _NP // 128


def _mesh():
    return plsc.VectorSubcoreMesh(core_axis_name="c", subcore_axis_name="s")


def _sc_params():
    cp = pltpu.CompilerParams()
    if "needs_layout_passes" in pltpu.CompilerParams.__dataclass_fields__:
        cp = dataclasses.replace(cp, needs_layout_passes=False)
    return cp


def _wid():
    return lax.axis_index("s") * _NC + lax.axis_index("c")


def _al8(v):
    return pl.multiple_of(v, 8)


def _attn_body(heads, C, xl_hbm, xr_hbm, srcp_hbm, dstp_hbm, att_hbm,
               alpha_hbm, part_hbm,
               src_all, dst_all, il0, ir0, il1, ir1, xl0, xr0, xl1, xr1,
               al_all, att_v, ath_v, loc_v, sem0, sem1):
    nsl = C // 16
    wid = _wid()
    ebase = wid * _EPW
    pltpu.sync_copy(att_hbm, att_v)
    pltpu.sync_copy(srcp_hbm.at[pl.ds(_al8(ebase), _NP)], src_all)
    pltpu.sync_copy(dstp_hbm.at[pl.ds(_al8(ebase), _NP)], dst_all)
    lane0 = lax.iota(jnp.int32, 16) == 0
    ci0 = lax.iota(jnp.int32, 16)

    @pl.loop(0, heads)
    def _h(h):
        for cs in range(nsl):
            ath_v[pl.ds(cs * 16, 16)] = att_v[pl.ds(_al8(h * C) + cs * 16, 16)]

        @pl.loop(0, _NP, step=16)
        def _z(i):
            loc_v[pl.ds(i, 16)] = jnp.full((16,), -jnp.inf, jnp.float32)

        def fire(ch, il, ir, xl, xr, sem):
            base = ch * _B

            @pl.loop(0, _B, step=16)
            def _i(i):
                sl = pl.ds(base + i, 16)
                il[pl.ds(i, 16)] = src_all[sl] * heads + h
                ir[pl.ds(i, 16)] = dst_all[sl] * heads + h

            pltpu.async_copy(xl_hbm.at[il], xl, sem)
            pltpu.async_copy(xr_hbm.at[ir], xr, sem)

        def crunch(ch, il, ir, xl, xr, sem):
            pltpu.make_async_copy(xl_hbm.at[il], xl, sem).wait()
            pltpu.make_async_copy(xr_hbm.at[ir], xr, sem).wait()
            base = ch * _B

            @pl.loop(0, _B, unroll=2)
            def _e(e):
                er = jnp.full((16,), e, jnp.int32)
                acc = jnp.zeros((16,), jnp.float32)
                for cs in range(nsl):
                    ci = ci0 + cs * 16
                    t = (plsc.load_gather(xl, [er, ci])
                         + plsc.load_gather(xr, [er, ci]))
                    t = jnp.maximum(t, 0.2 * t)
                    acc = acc + t * ath_v[pl.ds(cs * 16, 16)]
                s = jnp.sum(acc)
                plsc.store_scatter(al_all,
                                   [jnp.full((16,), base + e, jnp.int32)],
                                   jnp.full((16,), s, jnp.float32),
                                   mask=lane0)

            @pl.loop(0, _B, step=16)
            def _m(i):
                sl = pl.ds(base + i, 16)
                idx = dst_all[sl]
                val = al_all[sl]
                cur = plsc.load_gather(loc_v, [idx])
                plsc.store_scatter(loc_v, [idx], jnp.maximum(cur, val))
                chk = plsc.load_gather(loc_v, [idx])

                @pl.when(jnp.any(chk < val))
                def _fix():
                    for _ in range(15):
                        c2 = plsc.load_gather(loc_v, [idx])
                        plsc.store_scatter(loc_v, [idx],
                                           jnp.maximum(c2, val),
                                           mask=c2 < val)

        fire(0, il0, ir0, xl0, xr0, sem0)

        @pl.loop(0, _NCH - 1, step=2)
        def _p(ch):
            fire(ch + 1, il1, ir1, xl1, xr1, sem1)
            crunch(ch, il0, ir0, xl0, xr0, sem0)
            fire(ch + 2, il0, ir0, xl0, xr0, sem0)
            crunch(ch + 1, il1, ir1, xl1, xr1, sem1)

        crunch(_NCH - 1, il0, ir0, xl0, xr0, sem0)

        off = _al8((h * _NW + wid) * _NP)
        pltpu.sync_copy(al_all, alpha_hbm.at[pl.ds(off, _NP)])
        pltpu.sync_copy(loc_v, part_hbm.at[pl.ds(off, _NP)])


def _run_attn(heads, xl2d, xr2d, srcp, dstp, att_flat):
    C = xl2d.shape[1]
    kern = pl.kernel(
        functools.partial(_attn_body, heads, C),
        out_type=[jax.ShapeDtypeStruct((heads * _NW * _NP,), jnp.float32),
                  jax.ShapeDtypeStruct((heads * _NW * _NP,), jnp.float32)],
        mesh=_mesh(),
        compiler_params=_sc_params(),
        scratch_types=[
            pltpu.VMEM((_NP,), jnp.int32),
            pltpu.VMEM((_NP,), jnp.int32),
            pltpu.VMEM((_B,), jnp.int32),
            pltpu.VMEM((_B,), jnp.int32),
            pltpu.VMEM((_B,), jnp.int32),
            pltpu.VMEM((_B,), jnp.int32),
            pltpu.VMEM((_B, C), jnp.float32),
            pltpu.VMEM((_B, C), jnp.float32),
            pltpu.VMEM((_B, C), jnp.float32),
            pltpu.VMEM((_B, C), jnp.float32),
            pltpu.VMEM((_NP,), jnp.float32),
            pltpu.VMEM((heads * C,), jnp.float32),
            pltpu.VMEM((C,), jnp.float32),
            pltpu.VMEM((_NP,), jnp.float32),
            pltpu.SemaphoreType.DMA,
            pltpu.SemaphoreType.DMA,
        ],
    )
    return kern(xl2d, xr2d, srcp, dstp, att_flat)


def _attn_pair_body(heads, C, xl_hbm, xr_hbm, srcp_hbm, dstp_hbm, att_hbm,
                    alpha_hbm, part_hbm,
                    src_all, dst_all, il0, ir0, il1, ir1, xl0, xr0, xl1, xr1,
                    al_a, al_b, att_v, ath_v, loc_a, loc_b, sem0, sem1):
    hp_n = heads // 2
    ns4 = C // 32
    wid = _wid()
    ebase = wid * _EPW
    pltpu.sync_copy(att_hbm, att_v)
    pltpu.sync_copy(srcp_hbm.at[pl.ds(_al8(ebase), _NP)], src_all)
    pltpu.sync_copy(dstp_hbm.at[pl.ds(_al8(ebase), _NP)], dst_all)
    lane0 = lax.iota(jnp.int32, 16) == 0
    ci0 = lax.iota(jnp.int32, 16)

    @pl.loop(0, hp_n)
    def _h(hp):
        for cs in range(2 * C // 16):
            ath_v[pl.ds(cs * 16, 16)] = att_v[
                pl.ds(_al8(hp * 2 * C) + cs * 16, 16)]

        @pl.loop(0, _NP, step=16)
        def _z(i):
            ninf = jnp.full((16,), -jnp.inf, jnp.float32)
            loc_a[pl.ds(i, 16)] = ninf
            loc_b[pl.ds(i, 16)] = ninf

        def fire(ch, il, ir, xl, xr, sem):
            base = ch * _B

            @pl.loop(0, _B, step=16)
            def _i(i):
                sl = pl.ds(base + i, 16)
                il[pl.ds(i, 16)] = src_all[sl] * hp_n + hp
                ir[pl.ds(i, 16)] = dst_all[sl] * hp_n + hp

            pltpu.async_copy(xl_hbm.at[il], xl, sem)
            pltpu.async_copy(xr_hbm.at[ir], xr, sem)

        def crunch(ch, il, ir, xl, xr, sem):
            pltpu.make_async_copy(xl_hbm.at[il], xl, sem).wait()
            pltpu.make_async_copy(xr_hbm.at[ir], xr, sem).wait()
            base = ch * _B

            @pl.loop(0, _B)
            def _e(e):
                er = jnp.full((16,), e, jnp.int32)
                acc_a = jnp.zeros((16,), jnp.float32)
                acc_b = jnp.zeros((16,), jnp.float32)
                for cs in range(2 * ns4):
                    ci = ci0 + cs * 16
                    ale, alo = plsc.unpack(
                        plsc.bitcast(plsc.load_gather(xl, [er, ci]),
                                     jnp.bfloat16),
                        format=plsc.PackFormat.INTERLEAVED)
                    are, aro = plsc.unpack(
                        plsc.bitcast(plsc.load_gather(xr, [er, ci]),
                                     jnp.bfloat16),
                        format=plsc.PackFormat.INTERLEAVED)
                    te = ale + are
                    te = jnp.maximum(te, 0.2 * te)
                    to = alo + aro
                    to = jnp.maximum(to, 0.2 * to)
                    if cs < ns4:
                        ae = ath_v[pl.ds(cs * 16, 16)]
                        ao = ath_v[pl.ds(C // 2 + cs * 16, 16)]
                        acc_a = acc_a + te * ae + to * ao
                    else:
                        ae = ath_v[pl.ds(C // 2 + cs * 16, 16)]
                        ao = ath_v[pl.ds(C + cs * 16, 16)]
                        acc_b = acc_b + te * ae + to * ao
                ei = jnp.full((16,), base + e, jnp.int32)
                plsc.store_scatter(al_a, [ei],
                                   jnp.full((16,), jnp.sum(acc_a),
                                            jnp.float32), mask=lane0)
                plsc.store_scatter(al_b, [ei],
                                   jnp.full((16,), jnp.sum(acc_b),
                                            jnp.float32), mask=lane0)

            def smax(loc_v, al_all):
                @pl.loop(0, _B, step=16)
                def _m(i):
                    sl = pl.ds(base + i, 16)
                    idx = dst_all[sl]
                    val = al_all[sl]
                    cur = plsc.load_gather(loc_v, [idx])
                    plsc.store_scatter(loc_v, [idx], jnp.maximum(cur, val))
                    chk = plsc.load_gather(loc_v, [idx])

                    @pl.when(jnp.any(chk < val))
                    def _fix():
                        for _ in range(15):
                            c2 = plsc.load_gather(loc_v, [idx])
                            plsc.store_scatter(loc_v, [idx],
                                               jnp.maximum(c2, val),
                                               mask=c2 < val)

            smax(loc_a, al_a)
            smax(loc_b, al_b)

        fire(0, il0, ir0, xl0, xr0, sem0)

        @pl.loop(0, _NCH - 1, step=2)
        def _p(ch):
            fire(ch + 1, il1, ir1, xl1, xr1, sem1)
            crunch(ch, il0, ir0, xl0, xr0, sem0)
            fire(ch + 2, il0, ir0, xl0, xr0, sem0)
            crunch(ch + 1, il1, ir1, xl1, xr1, sem1)

        crunch(_NCH - 1, il0, ir0, xl0, xr0, sem0)

        offa = _al8(((2 * hp) * _NW + wid) * _NP)
        offb = _al8(((2 * hp + 1) * _NW + wid) * _NP)
        pltpu.sync_copy(al_a, alpha_hbm.at[pl.ds(offa, _NP)])
        pltpu.sync_copy(al_b, alpha_hbm.at[pl.ds(offb, _NP)])
        pltpu.sync_copy(loc_a, part_hbm.at[pl.ds(offa, _NP)])
        pltpu.sync_copy(loc_b, part_hbm.at[pl.ds(offb, _NP)])


def _run_attn_pair(heads, xlp, xrp, srcp, dstp, att_de_flat):
    C = xlp.shape[1]
    kern = pl.kernel(
        functools.partial(_attn_pair_body, heads, C),
        out_type=[jax.ShapeDtypeStruct((heads * _NW * _NP,), jnp.float32),
                  jax.ShapeDtypeStruct((heads * _NW * _NP,), jnp.float32)],
        mesh=_mesh(),
        compiler_params=_sc_params(),
        scratch_types=[
            pltpu.VMEM((_NP,), jnp.int32),
            pltpu.VMEM((_NP,), jnp.int32),
            pltpu.VMEM((_B,), jnp.int32),
            pltpu.VMEM((_B,), jnp.int32),
            pltpu.VMEM((_B,), jnp.int32),
            pltpu.VMEM((_B,), jnp.int32),
            pltpu.VMEM((_B, C), jnp.int32),
            pltpu.VMEM((_B, C), jnp.int32),
            pltpu.VMEM((_B, C), jnp.int32),
            pltpu.VMEM((_B, C), jnp.int32),
            pltpu.VMEM((_NP,), jnp.float32),
            pltpu.VMEM((_NP,), jnp.float32),
            pltpu.VMEM((heads * C,), jnp.float32),
            pltpu.VMEM((2 * C,), jnp.float32),
            pltpu.VMEM((_NP,), jnp.float32),
            pltpu.VMEM((_NP,), jnp.float32),
            pltpu.SemaphoreType.DMA,
            pltpu.SemaphoreType.DMA,
        ],
    )
    return kern(xlp, xrp, srcp, dstp, att_de_flat)


def _exden_body(heads, alpha_hbm, dstp_hbm, amax_hbm, ex_hbm, part_hbm,
                dst_all, al_all, ex_all, amax_v, den_loc, tag_loc):
    wid = _wid()
    ebase = wid * _EPW
    pltpu.sync_copy(dstp_hbm.at[pl.ds(_al8(ebase), _NP)], dst_all)
    lanes = lax.iota(jnp.int32, 16)

    @pl.loop(0, heads)
    def _h(h):
        off = _al8((h * _NW + wid) * _NP)
        pltpu.sync_copy(amax_hbm.at[pl.ds(_al8(h * _NP), _NP)], amax_v)
        pltpu.sync_copy(alpha_hbm.at[pl.ds(off, _NP)], al_all)

        @pl.loop(0, _NP, step=16)
        def _z(i):
            den_loc[pl.ds(i, 16)] = jnp.zeros((16,), jnp.float32)

        @pl.loop(0, _EPW, step=16)
        def _i(i):
            sl = pl.ds(i, 16)
            d16 = dst_all[sl]
            am16 = plsc.load_gather(amax_v, [d16])
            val = jnp.exp(al_all[sl] - am16)
            ex_all[sl] = val
            plsc.store_scatter(tag_loc, [d16], lanes)
            wtag = plsc.load_gather(tag_loc, [d16])
            win = wtag == lanes
            cur = plsc.load_gather(den_loc, [d16])
            plsc.store_scatter(den_loc, [d16], cur + val, mask=win)

            @pl.when(jnp.any(~win))
            def _fix():
                pending = ~win
                for _ in range(15):
                    plsc.store_scatter(tag_loc, [d16], lanes, mask=pending)
                    rt = plsc.load_gather(tag_loc, [d16])
                    w2 = pending & (rt == lanes)
                    c2 = plsc.load_gather(den_loc, [d16])
                    plsc.store_scatter(den_loc, [d16], c2 + val, mask=w2)
                    pending = pending & ~w2

        for t in range(_EPW, _NP, 16):
            ex_all[pl.ds(t, 16)] = jnp.zeros((16,), jnp.float32)

        pltpu.sync_copy(ex_all, ex_hbm.at[pl.ds(off, _NP)])
        pltpu.sync_copy(den_loc, part_hbm.at[pl.ds(off, _NP)])


def _run_exden(heads, alpha, dstp, amax):
    kern = pl.kernel(
        functools.partial(_exden_body, heads),
        out_type=[jax.ShapeDtypeStruct((heads * _NW * _NP,), jnp.float32),
                  jax.ShapeDtypeStruct((heads * _NW * _NP,), jnp.float32)],
        mesh=_mesh(),
        compiler_params=_sc_params(),
        scratch_types=[
            pltpu.VMEM((_NP,), jnp.int32),
            pltpu.VMEM((_NP,), jnp.float32),
            pltpu.VMEM((_NP,), jnp.float32),
            pltpu.VMEM((_NP,), jnp.float32),
            pltpu.VMEM((_NP,), jnp.float32),
            pltpu.VMEM((_NP,), jnp.int32),
        ],
    )
    return kern(alpha, dstp, amax)


def _agg_body(heads, C, xl_hbm, ex_hbm, srcp_hbm, dstp_hbm, zer_hbm,
              out_hbm, src_all, dst_all, ex0, ex1, il0, il1, ds0, ds1,
              xl0, xl1, out_sh, sem0, sem1):
    nsl = C // 16
    wid = _wid()
    ebase = wid * _EPW
    sid = lax.axis_index("s")
    cid = lax.axis_index("c")
    pltpu.sync_copy(srcp_hbm.at[pl.ds(_al8(ebase), _NP)], src_all)
    pltpu.sync_copy(dstp_hbm.at[pl.ds(_al8(ebase), _NP)], dst_all)
    ci0 = lax.iota(jnp.int32, 16)

    @pl.loop(0, heads)
    def _h(h):
        @pl.when(sid == 0)
        def _z():
            pltpu.sync_copy(zer_hbm, out_sh)

        off = _al8((h * _NW + wid) * _NP)
        plsc.subcore_barrier()

        def fire(ch, il, dsb, exb, xl, sem):
            base = ch * _B

            @pl.loop(0, _B, step=16)
            def _i(i):
                sl = pl.ds(base + i, 16)
                il[pl.ds(i, 16)] = src_all[sl] * heads + h
                dsb[pl.ds(i, 16)] = dst_all[sl]

            pltpu.async_copy(ex_hbm.at[pl.ds(off + ch * _B, _B)], exb, sem)
            pltpu.async_copy(xl_hbm.at[il], xl, sem)

        def crunch(ch, il, dsb, exb, xl, sem):
            pltpu.make_async_copy(ex_hbm.at[pl.ds(off, _B)], exb, sem).wait()
            pltpu.make_async_copy(xl_hbm.at[il], xl, sem).wait()
            base = ch * _B

            @pl.loop(0, _B, step=16)
            def _e(i):
                av16 = exb[pl.ds(i, 16)]
                for j in range(16):
                    ae = av16[j]
                    er = jnp.full((16,), i + j, jnp.int32)
                    for cs in range(nsl):
                        ci = ci0 + cs * 16
                        v = plsc.load_gather(xl, [er, ci])
                        plsc.store_scatter(xl, [er, ci], v * ae)

            pltpu.sync_copy(xl, out_sh.at[dsb], add=True)

        fire(0, il0, ds0, ex0, xl0, sem0)

        @pl.loop(0, _NCH - 1, step=2)
        def _p(ch):
            fire(ch + 1, il1, ds1, ex1, xl1, sem1)
            crunch(ch, il0, ds0, ex0, xl0, sem0)
            fire(ch + 2, il0, ds0, ex0, xl0, sem0)
            crunch(ch + 1, il1, ds1, ex1, xl1, sem1)

        crunch(_NCH - 1, il0, ds0, ex0, xl0, sem0)

        plsc.subcore_barrier()

        @pl.when(sid == 0)
        def _w():
            pltpu.sync_copy(out_sh, out_hbm.at[cid, h])

        plsc.subcore_barrier()


def _run_agg(heads, xl2d, ex, srcp, dstp, zeros_nc):
    C = xl2d.shape[1]
    kern = pl.kernel(
        functools.partial(_agg_body, heads, C),
        out_type=jax.ShapeDtypeStruct((_NC, heads, _N, C), jnp.float32),
        mesh=_mesh(),
        compiler_params=_sc_params(),
        scratch_types=[
            pltpu.VMEM((_NP,), jnp.int32),
            pltpu.VMEM((_NP,), jnp.int32),
            pltpu.VMEM((_B,), jnp.float32),
            pltpu.VMEM((_B,), jnp.float32),
            pltpu.VMEM((_B,), jnp.int32),
            pltpu.VMEM((_B,), jnp.int32),
            pltpu.VMEM((_B,), jnp.int32),
            pltpu.VMEM((_B,), jnp.int32),
            pltpu.VMEM((_B, C), jnp.float32),
            pltpu.VMEM((_B, C), jnp.float32),
            pltpu.VMEM_SHARED((_N, C), jnp.float32),
            pltpu.SemaphoreType.DMA,
            pltpu.SemaphoreType.DMA,
        ],
    )
    return kern(xl2d, ex, srcp, dstp, zeros_nc)


def _tc_proj(x, wl, wr, bn):
    n, d = x.shape
    k = wl.shape[1]

    def body(x_ref, wl_ref, wr_ref, ol_ref, or_ref):
        xv = x_ref[...]
        ol_ref[...] = jnp.dot(xv, wl_ref[...],
                              preferred_element_type=jnp.float32)
        or_ref[...] = jnp.dot(xv, wr_ref[...],
                              preferred_element_type=jnp.float32)

    return pl.pallas_call(
        body,
        grid=(n // bn,),
        in_specs=[pl.BlockSpec((bn, d), lambda i: (i, 0)),
                  pl.BlockSpec((d, k), lambda i: (0, 0)),
                  pl.BlockSpec((d, k), lambda i: (0, 0))],
        out_specs=[pl.BlockSpec((bn, k), lambda i: (i, 0)),
                   pl.BlockSpec((bn, k), lambda i: (i, 0))],
        out_shape=[jax.ShapeDtypeStruct((n, k), jnp.float32)] * 2,
    )(x, wl, wr)


def _tc_mid(p0, p1, den3, b3, wl3, wr3, bn):
    heads, n, c = p0.shape
    k2 = wl3.shape[2]

    def body(p0_ref, p1_ref, d_ref, b_ref, wl_ref, wr_ref, ol_ref, or_ref):
        accl = jnp.zeros((bn, k2), jnp.float32)
        accr = jnp.zeros((bn, k2), jnp.float32)
        den = d_ref[...]
        for h in range(heads):
            dh = den[:, h][:, None] + 1e-16
            hv = (p0_ref[h] + p1_ref[h]) / dh + b_ref[h]
            hv = jnp.where(hv > 0, hv, jnp.exp(jnp.minimum(hv, 0.0)) - 1.0)
            accl += jnp.dot(hv, wl_ref[h], preferred_element_type=jnp.float32)
            accr += jnp.dot(hv, wr_ref[h], preferred_element_type=jnp.float32)
        ol_ref[...] = accl
        or_ref[...] = accr

    return pl.pallas_call(
        body,
        grid=(n // bn,),
        in_specs=[pl.BlockSpec((heads, bn, c), lambda i: (0, i, 0)),
                  pl.BlockSpec((heads, bn, c), lambda i: (0, i, 0)),
                  pl.BlockSpec((bn, heads), lambda i: (i, 0)),
                  pl.BlockSpec((heads, 1, c), lambda i: (0, 0, 0)),
                  pl.BlockSpec((heads, c, k2), lambda i: (0, 0, 0)),
                  pl.BlockSpec((heads, c, k2), lambda i: (0, 0, 0))],
        out_specs=[pl.BlockSpec((bn, k2), lambda i: (i, 0)),
                   pl.BlockSpec((bn, k2), lambda i: (i, 0))],
        out_shape=[jax.ShapeDtypeStruct((n, k2), jnp.float32)] * 2,
    )(p0, p1, den3, b3, wl3, wr3)


def _tc_merge(part_flat, heads, op):
    part = part_flat.reshape(heads, _NW, _NP)

    def body(p_ref, o_ref):
        if op == "max":
            o_ref[...] = jnp.max(p_ref[...], axis=1, keepdims=True)
        else:
            o_ref[...] = jnp.sum(p_ref[...], axis=1, keepdims=True)

    out = pl.pallas_call(
        body,
        grid=(heads,),
        in_specs=[pl.BlockSpec((1, _NW, _NP), lambda i: (i, 0, 0))],
        out_specs=pl.BlockSpec((1, 1, _NP), lambda i: (i, 0, 0)),
        out_shape=jax.ShapeDtypeStruct((heads, 1, _NP), jnp.float32),
    )(part)
    return out.reshape(heads * _NP)


def _tc_final(p0, p1, den3, b2, batch3, wfc, bfc, bn):
    n = p0.shape[0]
    nblk = n // bn

    def body(p0_ref, p1_ref, d_ref, b_ref, bt_ref, wfc_ref, bfc_ref,
             logits_ref, prob_ref, acc_ref, cnt_ref):
        i = pl.program_id(0)

        @pl.when(i == 0)
        def _init():
            acc_ref[...] = jnp.zeros_like(acc_ref)
            cnt_ref[...] = jnp.zeros_like(cnt_ref)

        dh = d_ref[...] + 1e-16
        hv = (p0_ref[...] + p1_ref[...]) / dh + b_ref[...]
        hv = jnp.where(hv > 0, hv, jnp.exp(jnp.minimum(hv, 0.0)) - 1.0)
        bt = bt_ref[0, 0, :]
        oh = (bt[:, None] == lax.broadcasted_iota(jnp.int32, (1, _G), 1)
              ).astype(jnp.float32)
        acc_ref[...] += lax.dot_general(
            oh, hv, (((0,), (0,)), ((), ())),
            preferred_element_type=jnp.float32)
        cnt_ref[...] += jnp.sum(oh, axis=0)[:, None]

        @pl.when(i == nblk - 1)
        def _fin():
            pooled = acc_ref[...] / jnp.maximum(cnt_ref[...], 1.0)
            logits = jnp.dot(pooled, wfc_ref[...],
                             preferred_element_type=jnp.float32) + bfc_ref[...]
            m = jnp.max(logits, axis=1, keepdims=True)
            ez = jnp.exp(logits - m)
            prob_ref[...] = ez / jnp.sum(ez, axis=1, keepdims=True)
            logits_ref[...] = logits

    return pl.pallas_call(
        body,
        grid=(nblk,),
        in_specs=[pl.BlockSpec((bn, _OUT), lambda i: (i, 0)),
                  pl.BlockSpec((bn, _OUT), lambda i: (i, 0)),
                  pl.BlockSpec((bn, 1), lambda i: (i, 0)),
                  pl.BlockSpec((1, _OUT), lambda i: (0, 0)),
                  pl.BlockSpec((1, 1, bn), lambda i: (i, 0, 0)),
                  pl.BlockSpec((_OUT, 2), lambda i: (0, 0)),
                  pl.BlockSpec((1, 2), lambda i: (0, 0))],
        out_specs=[pl.BlockSpec((_G, 2), lambda i: (0, 0)),
                   pl.BlockSpec((_G, 2), lambda i: (0, 0))],
        out_shape=[jax.ShapeDtypeStruct((_G, 2), jnp.float32)] * 2,
        scratch_shapes=[pltpu.VMEM((_G, _OUT), jnp.float32),
                        pltpu.VMEM((_G, _OUT), jnp.float32)],
    )(p0, p1, den3, b2, batch3, wfc, bfc)


def _gat_layer(heads, xl2d, xr2d, srcp, dstp, att, zeros_nc,
               xlp=None, xrp=None):
    if xlp is not None:
        alpha, amax_p = _run_attn_pair(heads, xlp, xrp, srcp, dstp,
                                       _deint(att).reshape(-1))
    else:
        alpha, amax_p = _run_attn(heads, xl2d, xr2d, srcp, dstp,
                                  att.reshape(-1))
    amax = _tc_merge(amax_p, heads, "max")
    ex, den_p = _run_exden(heads, alpha, dstp, amax)
    out_p = _run_agg(heads, xl2d, ex, srcp, dstp, zeros_nc)
    den = _tc_merge(den_p, heads, "sum")
    den_t = den.reshape(heads, _NP).T
    return out_p, den_t


def _pack(a):
    b = a.astype(jnp.bfloat16).reshape(-1, _HID, 2)
    return jax.lax.bitcast_convert_type(b, jnp.int32)


def _deint(att):
    return jnp.concatenate([att[:, 0::2], att[:, 1::2]], axis=1).reshape(-1)


def kernel(x, edge_index, batch, Wl1, Wr1, att1, b1, Wl2, Wr2, att2, b2,
           Wfc, bfc):
    src, dst = edge_index[0], edge_index[1]
    srcp = jnp.pad(src, (0, _NP - _EPW))
    dstp = jnp.pad(dst, (0, _NP - _EPW))
    zeros_nc = jnp.zeros((_N, _HID), jnp.float32)

    xl1, xr1 = _tc_proj(x, Wl1, Wr1, 2000)
    xl1g = xl1.reshape(_N * _HEADS, _HID)
    xr1g = xr1.reshape(_N * _HEADS, _HID)
    out1p, den1p = _gat_layer(_HEADS, xl1g, xr1g, srcp, dstp, att1, zeros_nc)

    xl2, xr2 = _tc_mid(out1p[0], out1p[1], den1p,
                       b1.reshape(_HEADS, 1, _HID),
                       Wl2.reshape(_HEADS, _HID, _OUT),
                       Wr2.reshape(_HEADS, _HID, _OUT), 2000)
    out2p, den2p = _gat_layer(1, xl2, xr2, srcp, dstp, att2, zeros_nc)

    logits, y_prob = _tc_final(out2p[0].reshape(_N, _OUT),
                               out2p[1].reshape(_N, _OUT),
                               den2p, b2.reshape(1, -1),
                               batch.reshape(_N // 1000, 1, 1000),
                               Wfc, bfc.reshape(1, 2), 1000)
    return (logits, y_prob)
